# baseline (speedup 1.0000x reference)
import functools

import jax
import jax.numpy as jnp
from jax.experimental import pallas as pl
from jax.experimental.pallas import tpu as pltpu

_VMEM_LIMIT = 56 * 1024 * 1024
_BF = jnp.bfloat16
_F8 = jnp.float8_e4m3fn
_ADJ_SCALE = 256.0
_INV_ADJ_SCALE = 1.0 / 256.0


def _mega_kernel(x_ref, adj_ref, w1_ref, w2_ref, w3_ref, wc1_ref, bc1_ref,
                 wc2_ref, bc2_ref, z_ref, c_ref, s1_ref, s2_ref, s3_ref,
                 a8_ref, *, tab, gab, tma, ga, tmc, gc):
    i = pl.program_id(0)

    @pl.when(i < gab)
    def _phase_a():
        rows = pl.ds(i * tab, tab)
        s1 = jnp.dot(x_ref[...].astype(_BF), w1_ref[...].astype(_BF),
                     preferred_element_type=jnp.float32)
        s1_ref[rows, :] = jnp.tanh(s1).astype(_BF)

    @pl.when((i >= gab) & (i < gab + ga))
    def _phase_b():
        rows = pl.ds((i - gab) * tma, tma)
        adjf = adj_ref[...]
        a8_ref[rows, :] = (adjf * _ADJ_SCALE).astype(_F8)
        z1 = jnp.dot(adjf.astype(_BF), s1_ref[...],
                     preferred_element_type=jnp.float32)
        s2 = jnp.dot(z1.astype(_BF), w2_ref[...].astype(_BF),
                     preferred_element_type=jnp.float32)
        s2_ref[rows, :] = jnp.tanh(s2).astype(_BF)

    @pl.when((i >= gab + ga) & (i < gab + ga + gc))
    def _phase_c():
        rows = pl.ds((i - gab - ga) * tmc, tmc)
        adjb = a8_ref[rows, :].astype(_BF)
        z2 = jnp.dot(adjb, s2_ref[...], preferred_element_type=jnp.float32)
        w3 = (w3_ref[...] * _INV_ADJ_SCALE).astype(_BF)
        s3 = jnp.dot(z2.astype(_BF), w3, preferred_element_type=jnp.float32)
        s3_ref[rows, :] = s3.astype(_BF)

    @pl.when(i >= gab + ga + gc)
    def _phase_d():
        rows = pl.ds((i - gab - ga - gc) * tmc, tmc)
        adjb = a8_ref[rows, :].astype(_BF)
        z = jnp.dot(adjb, s3_ref[...],
                    preferred_element_type=jnp.float32) * _INV_ADJ_SCALE
        z_ref[...] = z
        h = jnp.dot(z.astype(_BF), wc1_ref[...].astype(_BF),
                    preferred_element_type=jnp.float32) + bc1_ref[...]
        h = jnp.maximum(h, 0.0)
        logits = jnp.dot(h.astype(_BF), wc2_ref[...].astype(_BF),
                         preferred_element_type=jnp.float32) + bc2_ref[...]
        m = jnp.max(logits, axis=-1, keepdims=True)
        e = jnp.exp(logits - m)
        c_ref[...] = e * pl.reciprocal(jnp.sum(e, axis=-1, keepdims=True))


def _full_spec(shape):
    return pl.BlockSpec(shape, lambda i, _s=shape: tuple(0 for _ in _s))


def kernel(x, adj, w1, w2, w3, wc1, bc1, wc2, bc2):
    N, n_input = x.shape
    enc1, enc2, enc3 = w1.shape[1], w2.shape[1], w3.shape[1]
    nc = wc2.shape[1]
    tab = min(512, N)
    gab = pl.cdiv(N, tab)
    tma = min(512, N)
    ga = pl.cdiv(N, tma)
    tmc = min(2048, N)
    gc = pl.cdiv(N, tmc)
    grid = (gab + ga + 2 * gc,)

    z_igae, c = pl.pallas_call(
        functools.partial(_mega_kernel, tab=tab, gab=gab, tma=tma, ga=ga,
                          tmc=tmc, gc=gc),
        out_shape=(jax.ShapeDtypeStruct((N, enc3), jnp.float32),
                   jax.ShapeDtypeStruct((N, nc), jnp.float32)),
        grid=grid,
        in_specs=[
            pl.BlockSpec((tab, n_input), lambda i: (i * (i < gab), 0)),
            pl.BlockSpec((tma, N),
                         lambda i: ((i - gab) * ((i >= gab) & (i < gab + ga))
                                    + (ga - 1) * (i >= gab + ga), 0)),
            _full_spec((n_input, enc1)), _full_spec((enc1, enc2)),
            _full_spec((enc2, enc3)), _full_spec((enc3, enc3)),
            _full_spec((1, enc3)), _full_spec((enc3, nc)),
            _full_spec((1, nc)),
        ],
        out_specs=(
            pl.BlockSpec((tmc, enc3),
                         lambda i: ((i - (gab + ga + gc))
                                    * (i >= gab + ga + gc), 0)),
            pl.BlockSpec((tmc, nc),
                         lambda i: ((i - (gab + ga + gc))
                                    * (i >= gab + ga + gc), 0)),
        ),
        scratch_shapes=[pltpu.VMEM((N, enc1), _BF),
                        pltpu.VMEM((N, enc2), _BF),
                        pltpu.VMEM((N, enc3), _BF),
                        pltpu.VMEM((N, N), _F8)],
        compiler_params=pltpu.CompilerParams(
            dimension_semantics=("arbitrary",),
            vmem_limit_bytes=_VMEM_LIMIT),
    )(x, adj, w1, w2, w3, wc1, bc1, wc2, bc2)

    return z_igae, c
